# baseline (speedup 1.0000x reference)
import math

import jax
import jax.numpy as jnp
from jax.experimental import pallas as pl

_P = 0.001
_CLIP_LO = math.log(_P / (1.0 - _P))
_CLIP_HI = -_CLIP_LO
_BB = 512

_CM_END = 800
_W0_END = _CM_END + 16 * 128
_W1_END = _W0_END + 16 * 64
_W2_END = _W1_END + 16 * 8


def _tree_select(a3, d3):
    m0, m1, m2, m3 = (d3[0] > 0, d3[1] > 0, d3[2] > 0, d3[3] > 0)
    t = jnp.where(m3[None], a3[8:16], a3[0:8])
    t = jnp.where(m2[None], t[4:8], t[0:4])
    t = jnp.where(m1[None], t[2:4], t[0:2])
    t = jnp.where(m0, t[1], t[0])
    return jnp.clip(t, _CLIP_LO, _CLIP_HI)


def _set_row(h, r0, bias):
    row = jax.lax.broadcasted_iota(jnp.int32, h.shape, 0)
    return jnp.where(row == r0, jnp.broadcast_to(bias, h.shape), h)


def _dot_nk(a, b):
    return jax.lax.dot_general(a, b, (((1,), (1,)), ((), ())),
                               preferred_element_type=jnp.float32)


def _gln_body(x_ref, bb_ref, buf_ref, b0_ref, b1_ref, out_ref):
    x = x_ref[...]
    base = jnp.clip(x, _P, 1.0 - _P)
    l0 = jnp.log(base / (1.0 - base))
    col = jax.lax.broadcasted_iota(jnp.int32, l0.shape, 1)
    l0 = jnp.where(col == 0, jnp.broadcast_to(bb_ref[...], l0.shape), l0)

    xb = x.astype(jnp.bfloat16)
    d = _dot_nk(buf_ref[0:_CM_END], xb)
    d0 = d[0:512].reshape(4, 128, -1)
    d1 = d[512:768].reshape(4, 64, -1)
    d2 = d[768:800].reshape(4, 8, -1)

    a0 = _dot_nk(buf_ref[_CM_END:_W0_END],
                 l0.astype(jnp.bfloat16)).reshape(16, 128, -1)
    h0 = _tree_select(a0, d0)

    l1 = _set_row(h0, 127, b0_ref[...])
    a1 = jnp.dot(buf_ref[_W0_END:_W1_END, 0:128], l1.astype(jnp.bfloat16),
                 preferred_element_type=jnp.float32).reshape(16, 64, -1)
    h1 = _tree_select(a1, d1)

    l2 = _set_row(h1, 63, b1_ref[...])
    a2 = jnp.dot(buf_ref[_W1_END:_W2_END, 0:64], l2.astype(jnp.bfloat16),
                 preferred_element_type=jnp.float32).reshape(16, 8, -1)
    o = _tree_select(a2, d2)[0:1]
    out_ref[...] = jnp.transpose(o)


def _prep_mj(t, sp, width):
    t = jnp.transpose(t, (1, 0, 2))
    t = jnp.pad(t, ((0, 0), (0, sp - t.shape[1]), (0, width - t.shape[2])))
    return t.reshape(t.shape[0] * sp, width).astype(jnp.bfloat16)


def kernel(x, base_bias, cm0, w0, b0, cm1, w1, b1, cm2, w2):
    B = x.shape[0]
    bb = jnp.asarray(base_bias, jnp.float32).reshape(1, 1)
    b0s = b0.reshape(1, 1)
    b1s = b1.reshape(1, 1)

    buf = jnp.concatenate([
        _prep_mj(cm0[0], 128, 256), _prep_mj(cm1[0], 64, 256),
        _prep_mj(cm2[0], 8, 256), _prep_mj(w0[0], 128, 256),
        _prep_mj(jnp.roll(w1[0], -1, axis=2), 64, 256),
        _prep_mj(jnp.roll(w2[0], -1, axis=2), 8, 256),
    ], axis=0)

    def fixed(a):
        return pl.BlockSpec(a.shape, lambda i: (0, 0))

    out = pl.pallas_call(
        _gln_body,
        grid=(B // _BB,),
        in_specs=[
            pl.BlockSpec((_BB, x.shape[1]), lambda i: (i, 0)),
            fixed(bb), fixed(buf), fixed(b0s), fixed(b1s),
        ],
        out_specs=pl.BlockSpec((_BB, 1), lambda i: (i, 0)),
        out_shape=jax.ShapeDtypeStruct((B, 1), jnp.float32),
    )(x, bb, buf, b0s, b1s)
    return out[:, :, None]


# speedup vs baseline: 1.2453x; 1.2453x over previous
import math

import jax
import jax.numpy as jnp
from jax.experimental import pallas as pl

_P = 0.001
_CLIP_LO = math.log(_P / (1.0 - _P))
_CLIP_HI = -_CLIP_LO
_BB = 1024

_CM_END = 800
_W0_END = _CM_END + 16 * 128
_W1_END = _W0_END + 16 * 64
_W2_END = _W1_END + 16 * 8


def _tree_select(a3, d3):
    m0, m1, m2, m3 = (d3[0] > 0, d3[1] > 0, d3[2] > 0, d3[3] > 0)
    t = jnp.where(m3[None], a3[8:16], a3[0:8])
    t = jnp.where(m2[None], t[4:8], t[0:4])
    t = jnp.where(m1[None], t[2:4], t[0:2])
    t = jnp.where(m0, t[1], t[0])
    return jnp.clip(t, _CLIP_LO, _CLIP_HI)


def _set_row(h, r0, bias):
    row = jax.lax.broadcasted_iota(jnp.int32, h.shape, 0)
    return jnp.where(row == r0, jnp.broadcast_to(bias, h.shape), h)


def _dot_nk(a, b):
    return jax.lax.dot_general(a, b, (((1,), (1,)), ((), ())),
                               preferred_element_type=jnp.float32)


def _gln_body(x_ref, bb_ref, buf_ref, b0_ref, b1_ref, out_ref):
    x = x_ref[...]
    base = jnp.clip(x, _P, 1.0 - _P)
    l0 = jnp.log(base / (1.0 - base))
    col = jax.lax.broadcasted_iota(jnp.int32, l0.shape, 1)
    l0 = jnp.where(col == 0, jnp.broadcast_to(bb_ref[...], l0.shape), l0)

    xb = x.astype(jnp.bfloat16)
    d = _dot_nk(buf_ref[0:_CM_END], xb)
    d0 = d[0:512].reshape(4, 128, -1)
    d1 = d[512:768].reshape(4, 64, -1)
    d2 = d[768:800].reshape(4, 8, -1)

    a0 = _dot_nk(buf_ref[_CM_END:_W0_END],
                 l0.astype(jnp.bfloat16)).reshape(16, 128, -1)
    h0 = _tree_select(a0, d0)

    l1 = _set_row(h0, 127, b0_ref[...])
    a1 = jnp.dot(buf_ref[_W0_END:_W1_END, 0:128], l1.astype(jnp.bfloat16),
                 preferred_element_type=jnp.float32).reshape(16, 64, -1)
    h1 = _tree_select(a1, d1)

    l2 = _set_row(h1, 63, b1_ref[...])
    a2 = jnp.dot(buf_ref[_W1_END:_W2_END, 0:64], l2.astype(jnp.bfloat16),
                 preferred_element_type=jnp.float32).reshape(16, 8, -1)
    out_ref[...] = _tree_select(a2, d2)[0:1]


def _prep_mj(t, sp, width):
    t = jnp.transpose(t, (1, 0, 2))
    t = jnp.pad(t, ((0, 0), (0, sp - t.shape[1]), (0, width - t.shape[2])))
    return t.reshape(t.shape[0] * sp, width).astype(jnp.bfloat16)


def kernel(x, base_bias, cm0, w0, b0, cm1, w1, b1, cm2, w2):
    B = x.shape[0]
    bb = jnp.asarray(base_bias, jnp.float32).reshape(1, 1)
    b0s = b0.reshape(1, 1)
    b1s = b1.reshape(1, 1)

    buf = jnp.concatenate([
        _prep_mj(cm0[0], 128, 256), _prep_mj(cm1[0], 64, 256),
        _prep_mj(cm2[0], 8, 256), _prep_mj(w0[0], 128, 256),
        _prep_mj(jnp.roll(w1[0], -1, axis=2), 64, 256),
        _prep_mj(jnp.roll(w2[0], -1, axis=2), 8, 256),
    ], axis=0)

    def fixed(a):
        return pl.BlockSpec(a.shape, lambda i: (0, 0))

    out = pl.pallas_call(
        _gln_body,
        grid=(B // _BB,),
        in_specs=[
            pl.BlockSpec((_BB, x.shape[1]), lambda i: (i, 0)),
            fixed(bb), fixed(buf), fixed(b0s), fixed(b1s),
        ],
        out_specs=pl.BlockSpec((1, _BB), lambda i: (0, i)),
        out_shape=jax.ShapeDtypeStruct((1, B), jnp.float32),
    )(x, bb, buf, b0s, b1s)
    return out.reshape(B, 1, 1)
